# Initial kernel scaffold; baseline (speedup 1.0000x reference)
import functools

import jax
import jax.numpy as jnp
from jax import lax
from jax.experimental import pallas as pl
from jax.experimental.pallas import tpu as pltpu
from jax.experimental.pallas import tpu_sc as plsc

N = 50000
E = 800000
D = 64
K = 4
C = 10
G = 64
GAMMA = 0.01
NV = 100
EV = 5

NW = 32
BLK = 1568
NPAD = NW * BLK
HALF = NPAD // 2
EPW = E // NW
PC = 512
NCH_P = (EPW + PC - 1) // PC
CAP = 25216
CH = 128
AGG_ROWS = 26112
TRASH = HALF
HI = lax.Precision.HIGHEST

_mesh = plsc.VectorSubcoreMesh(core_axis_name="c", subcore_axis_name="s")


@functools.partial(
    pl.kernel,
    out_type=(
        jax.ShapeDtypeStruct((NW, 2, CAP), jnp.int32),
        jax.ShapeDtypeStruct((NW, 2, CAP), jnp.int32),
        jax.ShapeDtypeStruct((NW, 16), jnp.int32),
    ),
    mesh=_mesh,
    scratch_types=[
        pltpu.VMEM((PC,), jnp.int32),
        pltpu.VMEM((PC,), jnp.int32),
        pltpu.VMEM((PC,), jnp.int32),
        pltpu.VMEM((CAP,), jnp.int32),
        pltpu.VMEM((CAP,), jnp.int32),
        pltpu.VMEM((CAP,), jnp.int32),
        pltpu.VMEM((CAP,), jnp.int32),
        pltpu.VMEM((16,), jnp.int32),
    ],
)
def _partition(src_hbm, dst_hbm, attr_hbm, gidx_out, dstl_out, cnt_out,
               src_v, dst_v, attr_v, g0, d0, g1, d1, cnt_v):
    c = lax.axis_index("c")
    s = lax.axis_index("s")
    w = s * 2 + c
    base = w * EPW
    lane = lax.broadcasted_iota(jnp.int32, (16,), 0)

    def chunk_body(g, offs):
        off = base + g * PC
        pltpu.sync_copy(src_hbm.at[pl.ds(off, PC)], src_v)
        pltpu.sync_copy(dst_hbm.at[pl.ds(off, PC)], dst_v)
        pltpu.sync_copy(attr_hbm.at[pl.ds(off, PC)], attr_v)

        def vec_body(i, offs2):
            off0, off1 = offs2
            s16 = src_v[pl.ds(i * 16, 16)]
            d16 = dst_v[pl.ds(i * 16, 16)]
            a16 = attr_v[pl.ds(i * 16, 16)]
            gidx = a16 * NPAD + s16
            valid = (g * PC + i * 16 + lane) < EPW
            m0 = valid & (d16 < HALF)
            m1 = valid & (d16 >= HALF)
            mi0 = m0.astype(jnp.int32)
            mi1 = m1.astype(jnp.int32)
            cs0 = plsc.cumsum(mi0)
            cs1 = plsc.cumsum(mi1)
            pos0 = off0 + cs0 - mi0
            pos1 = off1 + cs1 - mi1
            plsc.store_scatter(g0, [pos0], gidx, mask=m0)
            plsc.store_scatter(d0, [pos0], d16, mask=m0)
            plsc.store_scatter(g1, [pos1], gidx, mask=m1)
            plsc.store_scatter(d1, [pos1], d16 - HALF, mask=m1)
            return (off0 + jnp.sum(mi0), off1 + jnp.sum(mi1))

        return lax.fori_loop(0, PC // 16, vec_body, offs)

    off0, off1 = lax.fori_loop(0, NCH_P, chunk_body,
                               (jnp.int32(0), jnp.int32(0)))

    zero16 = jnp.zeros((16,), jnp.int32)
    trash16 = jnp.full((16,), TRASH, jnp.int32)
    for j in range(CH // 16):
        g0[pl.ds(off0 + j * 16, 16)] = zero16
        d0[pl.ds(off0 + j * 16, 16)] = trash16
        g1[pl.ds(off1 + j * 16, 16)] = zero16
        d1[pl.ds(off1 + j * 16, 16)] = trash16

    pltpu.sync_copy(g0, gidx_out.at[w, 0])
    pltpu.sync_copy(d0, dstl_out.at[w, 0])
    pltpu.sync_copy(g1, gidx_out.at[w, 1])
    pltpu.sync_copy(d1, dstl_out.at[w, 1])
    cnt_v[...] = jnp.where(lane == 0, off0, jnp.where(lane == 1, off1, 0))
    pltpu.sync_copy(cnt_v, cnt_out.at[w])


@functools.partial(
    pl.kernel,
    out_type=jax.ShapeDtypeStruct((NPAD, D), jnp.float32),
    mesh=_mesh,
    scratch_types=[
        pltpu.VMEM_SHARED((AGG_ROWS, D), jnp.float32),
        pltpu.VMEM((CH, D), jnp.float32),
        pltpu.VMEM((CH,), jnp.int32),
        pltpu.VMEM((CH,), jnp.int32),
        pltpu.VMEM((16,), jnp.int32),
    ],
)
def _aggregate(m_hbm, gidx_hbm, dstl_hbm, cnt_hbm, zeros_hbm, agg_out,
               agg_sh, gbuf, gidx_v, dstl_v, cnt_v):
    c = lax.axis_index("c")
    s = lax.axis_index("s")
    lane = lax.broadcasted_iota(jnp.int32, (16,), 0)

    pltpu.sync_copy(zeros_hbm.at[pl.ds(s * (AGG_ROWS // 16), AGG_ROWS // 16)],
                    agg_sh.at[pl.ds(s * (AGG_ROWS // 16), AGG_ROWS // 16)])
    plsc.subcore_barrier()

    for k in range(2):
        w = s * 2 + k
        pltpu.sync_copy(cnt_hbm.at[w], cnt_v)
        cnt = jnp.max(jnp.where(lane == c, cnt_v[...], 0))
        nch = (cnt + (CH - 1)) // CH

        def body(g, _):
            pltpu.sync_copy(gidx_hbm.at[w, c, pl.ds(g * CH, CH)], gidx_v)
            pltpu.sync_copy(dstl_hbm.at[w, c, pl.ds(g * CH, CH)], dstl_v)
            pltpu.sync_copy(m_hbm.at[gidx_v], gbuf)
            pltpu.sync_copy(gbuf, agg_sh.at[dstl_v], add=True)
            return 0

        lax.fori_loop(0, nch, body, 0)

    plsc.subcore_barrier()
    pltpu.sync_copy(agg_sh.at[pl.ds(s * BLK, BLK)],
                    agg_out.at[pl.ds(c * HALF + s * BLK, BLK)])


def _embed_body(x_ref, emb_ref, o_ref):
    xv = x_ref[...].reshape(1, BLK)
    oh = (lax.broadcasted_iota(jnp.int32, (NV, BLK), 0) == xv).astype(jnp.float32)
    o_ref[...] = lax.dot_general(oh, emb_ref[...], (((0,), (0,)), ((), ())),
                                 precision=HI)


def _embed(x3d, node_emb):
    return pl.pallas_call(
        _embed_body,
        grid=(NW,),
        in_specs=[
            pl.BlockSpec((1, 1, BLK), lambda b: (b, 0, 0)),
            pl.BlockSpec((NV, D), lambda b: (0, 0)),
        ],
        out_specs=pl.BlockSpec((BLK, D), lambda b: (b, 0)),
        out_shape=jax.ShapeDtypeStruct((NPAD, D), jnp.float32),
    )(x3d, node_emb)


def _mtable_body(h_ref, e_ref, o_ref):
    a = pl.program_id(0)
    erow = e_ref[...][a]
    o_ref[...] = jnp.maximum(h_ref[...] + erow[None, :], 0.0)[None]


def _mtable(h, eemb):
    m3 = pl.pallas_call(
        _mtable_body,
        grid=(EV, NW),
        in_specs=[
            pl.BlockSpec((BLK, D), lambda a, b: (b, 0)),
            pl.BlockSpec((EV, D), lambda a, b: (0, 0)),
        ],
        out_specs=pl.BlockSpec((1, BLK, D), lambda a, b: (a, b, 0)),
        out_shape=jax.ShapeDtypeStruct((EV, NPAD, D), jnp.float32),
    )(h, eemb)
    return m3.reshape(EV * NPAD, D)


def _mlp_body(h_ref, a_ref, w1_ref, b1_ref, w2_ref, b2_ref, ep_ref, o_ref,
              *, last):
    z = h_ref[...] * ep_ref[0, 0] + a_ref[...]
    z1 = lax.dot_general(z, w1_ref[...], (((1,), (1,)), ((), ())),
                         precision=HI) + b1_ref[0:1, :]
    z1 = jnp.maximum(z1, 0.0)
    z2 = lax.dot_general(z1, w2_ref[...], (((1,), (1,)), ((), ())),
                         precision=HI) + b2_ref[0:1, :]
    o_ref[...] = z2 if last else jnp.maximum(z2, 0.0)


def _mlp(h, agg, w1, b1bc, w2, b2bc, epbc, last):
    return pl.pallas_call(
        functools.partial(_mlp_body, last=last),
        grid=(NW,),
        in_specs=[
            pl.BlockSpec((BLK, D), lambda b: (b, 0)),
            pl.BlockSpec((BLK, D), lambda b: (b, 0)),
            pl.BlockSpec((2 * D, D), lambda b: (0, 0)),
            pl.BlockSpec((8, 2 * D), lambda b: (0, 0)),
            pl.BlockSpec((D, 2 * D), lambda b: (0, 0)),
            pl.BlockSpec((8, D), lambda b: (0, 0)),
            pl.BlockSpec((8, 128), lambda b: (0, 0)),
        ],
        out_specs=pl.BlockSpec((BLK, D), lambda b: (b, 0)),
        out_shape=jax.ShapeDtypeStruct((NPAD, D), jnp.float32),
    )(h, agg, w1, b1bc, w2, b2bc, epbc)


def _readout_body(h_ref, bt_ref, p_ref, wp_ref, bp_ref, o_ref):
    b = pl.program_id(0)
    hh = h_ref[...]
    t = lax.dot_general(hh, p_ref[...], (((1,), (1,)), ((), ())),
                        precision=HI) * (1.0 / GAMMA)
    t = t - jnp.max(t, axis=1, keepdims=True)
    ex = jnp.exp(t)
    al = ex / jnp.sum(ex, axis=1, keepdims=True)
    w2 = jnp.concatenate([al[:, k:k + 1] * hh for k in range(K)], axis=1)
    ws = lax.dot_general(w2, wp_ref[...], (((1,), (1,)), ((), ())),
                         precision=HI)
    bv = bt_ref[...].reshape(1, BLK)
    oh = (lax.broadcasted_iota(jnp.int32, (G, BLK), 0) == bv).astype(jnp.float32)
    contrib = lax.dot_general(oh, ws, (((1,), (0,)), ((), ())), precision=HI)

    @pl.when(b == 0)
    def _():
        o_ref[...] = contrib + bp_ref[0:1, :]

    @pl.when(b > 0)
    def _():
        o_ref[...] = o_ref[...] + contrib


def _readout(h, batch3d, p, wp, bpbc):
    return pl.pallas_call(
        _readout_body,
        grid=(NW,),
        in_specs=[
            pl.BlockSpec((BLK, D), lambda b: (b, 0)),
            pl.BlockSpec((1, 1, BLK), lambda b: (b, 0, 0)),
            pl.BlockSpec((K, D), lambda b: (0, 0)),
            pl.BlockSpec((C, K * D), lambda b: (0, 0)),
            pl.BlockSpec((8, C), lambda b: (0, 0)),
        ],
        out_specs=pl.BlockSpec((G, C), lambda b: (0, 0)),
        out_shape=jax.ShapeDtypeStruct((G, C), jnp.float32),
    )(h, batch3d, p, wp, bpbc)


def kernel(x, edge_index, edge_attr, batch, node_emb, edge_embs,
           W1, b1, W2, b2, eps, P, Wp, bp):
    L = W1.shape[0]
    pad_e = NCH_P * PC - E
    src = jnp.pad(edge_index[0].astype(jnp.int32), (0, pad_e))
    dst = jnp.pad(edge_index[1].astype(jnp.int32), (0, pad_e))
    attr = jnp.pad(edge_attr.astype(jnp.int32), (0, pad_e))
    x3d = jnp.pad(x.astype(jnp.int32), (0, NPAD - N)).reshape(NW, 1, BLK)
    batch3d = jnp.pad(batch.astype(jnp.int32), (0, NPAD - N),
                      constant_values=G).reshape(NW, 1, BLK)
    zeros_hbm = jnp.zeros((AGG_ROWS, D), jnp.float32)
    b1bc = jnp.broadcast_to(b1.reshape(L, 1, 2 * D), (L, 8, 2 * D))
    b2bc = jnp.broadcast_to(b2.reshape(L, 1, D), (L, 8, D))
    bpbc = jnp.broadcast_to(bp.reshape(1, C), (8, C))

    gidx, dstl, cnts = _partition(src, dst, attr)
    h = _embed(x3d, node_emb)
    for l in range(L):
        m2 = _mtable(h, edge_embs[l])
        agg = _aggregate(m2, gidx, dstl, cnts, zeros_hbm)
        epbc = jnp.full((8, 128), 1.0 + eps[l], jnp.float32)
        h = _mlp(h, agg, W1[l], b1bc[l], W2[l], b2bc[l], epbc, last=(l == L - 1))
    return _readout(h, batch3d, P, Wp, bpbc)


# speedup vs baseline: 4.2688x; 4.2688x over previous
import functools

import jax
import jax.numpy as jnp
from jax import lax
from jax.experimental import pallas as pl
from jax.experimental.pallas import tpu as pltpu
from jax.experimental.pallas import tpu_sc as plsc

N = 50000
E = 800000
D = 64
K = 4
C = 10
G = 64
GAMMA = 0.01
NV = 100
EV = 5

NW = 32
BLK = 1568
NPAD = NW * BLK
HALF = NPAD // 2
EPW = E // NW
PC = 512
NCH_P = (EPW + PC - 1) // PC
CAP = 25216
CH = 128
AGG_ROWS = 26112
TRASH = HALF
HI = lax.Precision.HIGHEST

_mesh = plsc.VectorSubcoreMesh(core_axis_name="c", subcore_axis_name="s")


def _vlast(v):
    return lax.squeeze(lax.slice(v, (15,), (16,)), (0,))


@functools.partial(
    pl.kernel,
    out_type=(
        jax.ShapeDtypeStruct((NW, 2, CAP), jnp.int32),
        jax.ShapeDtypeStruct((NW, 2, CAP), jnp.int32),
        jax.ShapeDtypeStruct((NW, 16), jnp.int32),
    ),
    mesh=_mesh,
    compiler_params=pltpu.CompilerParams(needs_layout_passes=False),
    scratch_types=[
        pltpu.VMEM((PC,), jnp.int32),
        pltpu.VMEM((PC,), jnp.int32),
        pltpu.VMEM((PC,), jnp.int32),
        pltpu.VMEM((CAP,), jnp.int32),
        pltpu.VMEM((CAP,), jnp.int32),
        pltpu.VMEM((CAP,), jnp.int32),
        pltpu.VMEM((CAP,), jnp.int32),
        pltpu.VMEM((16,), jnp.int32),
    ],
)
def _partition(src_hbm, dst_hbm, attr_hbm, gidx_out, dstl_out, cnt_out,
               src_v, dst_v, attr_v, g0, d0, g1, d1, cnt_v):
    c = lax.axis_index("c")
    s = lax.axis_index("s")
    w = s * 2 + c
    base = w * EPW
    lane = lax.broadcasted_iota(jnp.int32, (16,), 0)

    def chunk_body(g, offs):
        off = base + g * PC
        pltpu.sync_copy(src_hbm.at[pl.ds(off, PC)], src_v)
        pltpu.sync_copy(dst_hbm.at[pl.ds(off, PC)], dst_v)
        pltpu.sync_copy(attr_hbm.at[pl.ds(off, PC)], attr_v)

        npad16 = jnp.full((16,), NPAD, jnp.int32)
        half16 = jnp.full((16,), HALF, jnp.int32)
        epw16 = jnp.full((16,), EPW, jnp.int32)
        one16 = jnp.full((16,), 1, jnp.int32)
        zro16 = jnp.full((16,), 0, jnp.int32)

        def vec_body(i, offs2):
            off0, off1 = offs2
            s16 = src_v[pl.ds(i * 16, 16)]
            d16 = dst_v[pl.ds(i * 16, 16)]
            a16 = attr_v[pl.ds(i * 16, 16)]
            gidx = a16 * npad16 + s16
            pvec = lax.broadcast_in_dim(g * PC + i * 16, (16,), ()) + lane
            valid = pvec < epw16
            m0 = valid & (d16 < half16)
            m1 = valid & (d16 >= half16)
            mi0 = jnp.where(m0, one16, zro16)
            mi1 = jnp.where(m1, one16, zro16)
            cs0 = plsc.cumsum(mi0)
            cs1 = plsc.cumsum(mi1)
            pos0 = lax.broadcast_in_dim(off0, (16,), ()) + cs0 - mi0
            pos1 = lax.broadcast_in_dim(off1, (16,), ()) + cs1 - mi1
            plsc.store_scatter(g0, [pos0], gidx, mask=m0)
            plsc.store_scatter(d0, [pos0], d16, mask=m0)
            plsc.store_scatter(g1, [pos1], gidx, mask=m1)
            plsc.store_scatter(d1, [pos1], d16 - half16, mask=m1)
            return (off0 + _vlast(cs0), off1 + _vlast(cs1))

        return lax.fori_loop(0, PC // 16, vec_body, offs)

    off0, off1 = lax.fori_loop(0, NCH_P, chunk_body,
                               (jnp.int32(0), jnp.int32(0)))

    zero16 = jnp.zeros((16,), jnp.int32)
    trash16 = jnp.full((16,), TRASH, jnp.int32)
    for j in range(CH // 16):
        g0[pl.ds(off0 + j * 16, 16)] = zero16
        d0[pl.ds(off0 + j * 16, 16)] = trash16
        g1[pl.ds(off1 + j * 16, 16)] = zero16
        d1[pl.ds(off1 + j * 16, 16)] = trash16

    pltpu.sync_copy(g0, gidx_out.at[w, 0])
    pltpu.sync_copy(d0, dstl_out.at[w, 0])
    pltpu.sync_copy(g1, gidx_out.at[w, 1])
    pltpu.sync_copy(d1, dstl_out.at[w, 1])
    zv = jnp.zeros((16,), jnp.int32)
    cnt_v[...] = jnp.where(lane == zv, lax.broadcast_in_dim(off0, (16,), ()),
                           jnp.where(lane == zv + 1,
                                     lax.broadcast_in_dim(off1, (16,), ()), zv))
    pltpu.sync_copy(cnt_v, cnt_out.at[w])


@functools.partial(
    pl.kernel,
    out_type=jax.ShapeDtypeStruct((NPAD, D), jnp.float32),
    mesh=_mesh,
    compiler_params=pltpu.CompilerParams(needs_layout_passes=False,
                                         use_tc_tiling_on_sc=False),
    scratch_types=[
        pltpu.VMEM_SHARED((AGG_ROWS, D), jnp.float32),
        pltpu.VMEM((CH, D), jnp.float32),
        pltpu.VMEM((CH,), jnp.int32),
        pltpu.VMEM((CH,), jnp.int32),
        pltpu.VMEM((16,), jnp.int32),
    ],
)
def _aggregate(m_hbm, gidx_hbm, dstl_hbm, cnt_hbm, zeros_hbm, agg_out,
               agg_sh, gbuf, gidx_v, dstl_v, cnt_v):
    c = lax.axis_index("c")
    s = lax.axis_index("s")
    lane = lax.broadcasted_iota(jnp.int32, (16,), 0)

    pltpu.sync_copy(zeros_hbm.at[pl.ds(s * (AGG_ROWS // 16), AGG_ROWS // 16)],
                    agg_sh.at[pl.ds(s * (AGG_ROWS // 16), AGG_ROWS // 16)])
    plsc.subcore_barrier()

    for k in range(2):
        w = s * 2 + k
        pltpu.sync_copy(cnt_hbm.at[w], cnt_v)
        cvec = lax.broadcast_in_dim(c, (16,), ())
        sel = jnp.where(lane == cvec, cnt_v[...], jnp.zeros((16,), jnp.int32))
        cnt = _vlast(plsc.cumsum(sel))
        nch = (cnt + (CH - 1)) // CH

        def body(g, _):
            pltpu.sync_copy(gidx_hbm.at[w, c, pl.ds(g * CH, CH)], gidx_v)
            pltpu.sync_copy(dstl_hbm.at[w, c, pl.ds(g * CH, CH)], dstl_v)
            pltpu.sync_copy(m_hbm.at[gidx_v], gbuf)
            pltpu.sync_copy(gbuf, agg_sh.at[dstl_v], add=True)
            return 0

        lax.fori_loop(0, nch, body, 0)

    plsc.subcore_barrier()
    pltpu.sync_copy(agg_sh.at[pl.ds(s * BLK, BLK)],
                    agg_out.at[pl.ds(c * HALF + s * BLK, BLK)])


def _embed_body(x_ref, emb_ref, o_ref):
    xv = x_ref[...].reshape(1, BLK)
    oh = (lax.broadcasted_iota(jnp.int32, (NV, BLK), 0) == xv).astype(jnp.float32)
    o_ref[...] = lax.dot_general(oh, emb_ref[...], (((0,), (0,)), ((), ())),
                                 precision=HI)


def _embed(x3d, node_emb):
    return pl.pallas_call(
        _embed_body,
        grid=(NW,),
        in_specs=[
            pl.BlockSpec((1, 1, BLK), lambda b: (b, 0, 0)),
            pl.BlockSpec((NV, D), lambda b: (0, 0)),
        ],
        out_specs=pl.BlockSpec((BLK, D), lambda b: (b, 0)),
        out_shape=jax.ShapeDtypeStruct((NPAD, D), jnp.float32),
    )(x3d, node_emb)


def _mtable_body(h_ref, e_ref, o_ref):
    a = pl.program_id(0)
    sel = (lax.broadcasted_iota(jnp.int32, (EV, 1), 0) == a).astype(jnp.float32)
    erow = jnp.sum(e_ref[...] * sel, axis=0, keepdims=True)
    o_ref[...] = jnp.maximum(h_ref[...] + erow, 0.0)[None]


def _mtable(h, eemb):
    m3 = pl.pallas_call(
        _mtable_body,
        grid=(EV, NW),
        in_specs=[
            pl.BlockSpec((BLK, D), lambda a, b: (b, 0)),
            pl.BlockSpec((EV, D), lambda a, b: (0, 0)),
        ],
        out_specs=pl.BlockSpec((1, BLK, D), lambda a, b: (a, b, 0)),
        out_shape=jax.ShapeDtypeStruct((EV, NPAD, D), jnp.float32),
    )(h, eemb)
    return m3.reshape(EV * NPAD, D)


def _mlp_body(h_ref, a_ref, w1_ref, b1_ref, w2_ref, b2_ref, ep_ref, o_ref,
              *, last):
    z = h_ref[...] * ep_ref[0, 0] + a_ref[...]
    z1 = lax.dot_general(z, w1_ref[...], (((1,), (1,)), ((), ())),
                         precision=HI) + b1_ref[0:1, :]
    z1 = jnp.maximum(z1, 0.0)
    z2 = lax.dot_general(z1, w2_ref[...], (((1,), (1,)), ((), ())),
                         precision=HI) + b2_ref[0:1, :]
    o_ref[...] = z2 if last else jnp.maximum(z2, 0.0)


def _mlp(h, agg, w1, b1bc, w2, b2bc, epbc, last):
    return pl.pallas_call(
        functools.partial(_mlp_body, last=last),
        grid=(NW,),
        in_specs=[
            pl.BlockSpec((BLK, D), lambda b: (b, 0)),
            pl.BlockSpec((BLK, D), lambda b: (b, 0)),
            pl.BlockSpec((2 * D, D), lambda b: (0, 0)),
            pl.BlockSpec((8, 2 * D), lambda b: (0, 0)),
            pl.BlockSpec((D, 2 * D), lambda b: (0, 0)),
            pl.BlockSpec((8, D), lambda b: (0, 0)),
            pl.BlockSpec((8, 128), lambda b: (0, 0)),
        ],
        out_specs=pl.BlockSpec((BLK, D), lambda b: (b, 0)),
        out_shape=jax.ShapeDtypeStruct((NPAD, D), jnp.float32),
    )(h, agg, w1, b1bc, w2, b2bc, epbc)


def _readout_body(h_ref, bt_ref, p_ref, wp_ref, bp_ref, o_ref):
    b = pl.program_id(0)
    hh = h_ref[...]
    t = lax.dot_general(hh, p_ref[...], (((1,), (1,)), ((), ())),
                        precision=HI) * (1.0 / GAMMA)
    t = t - jnp.max(t, axis=1, keepdims=True)
    ex = jnp.exp(t)
    al = ex / jnp.sum(ex, axis=1, keepdims=True)
    w2 = jnp.concatenate([al[:, k:k + 1] * hh for k in range(K)], axis=1)
    ws = lax.dot_general(w2, wp_ref[...], (((1,), (1,)), ((), ())),
                         precision=HI)
    bv = bt_ref[...].reshape(1, BLK)
    oh = (lax.broadcasted_iota(jnp.int32, (G, BLK), 0) == bv).astype(jnp.float32)
    contrib = lax.dot_general(oh, ws, (((1,), (0,)), ((), ())), precision=HI)

    @pl.when(b == 0)
    def _():
        o_ref[...] = contrib + bp_ref[0:1, :]

    @pl.when(b > 0)
    def _():
        o_ref[...] = o_ref[...] + contrib


def _readout(h, batch3d, p, wp, bpbc):
    return pl.pallas_call(
        _readout_body,
        grid=(NW,),
        in_specs=[
            pl.BlockSpec((BLK, D), lambda b: (b, 0)),
            pl.BlockSpec((1, 1, BLK), lambda b: (b, 0, 0)),
            pl.BlockSpec((K, D), lambda b: (0, 0)),
            pl.BlockSpec((C, K * D), lambda b: (0, 0)),
            pl.BlockSpec((8, C), lambda b: (0, 0)),
        ],
        out_specs=pl.BlockSpec((G, C), lambda b: (0, 0)),
        out_shape=jax.ShapeDtypeStruct((G, C), jnp.float32),
    )(h, batch3d, p, wp, bpbc)


def kernel(x, edge_index, edge_attr, batch, node_emb, edge_embs,
           W1, b1, W2, b2, eps, P, Wp, bp):
    L = W1.shape[0]
    pad_e = NCH_P * PC - EPW
    src = jnp.pad(edge_index[0].astype(jnp.int32), (0, pad_e))
    dst = jnp.pad(edge_index[1].astype(jnp.int32), (0, pad_e))
    attr = jnp.pad(edge_attr.astype(jnp.int32), (0, pad_e))
    x3d = jnp.pad(x.astype(jnp.int32), (0, NPAD - N)).reshape(NW, 1, BLK)
    batch3d = jnp.pad(batch.astype(jnp.int32), (0, NPAD - N),
                      constant_values=G).reshape(NW, 1, BLK)
    zeros_hbm = jnp.zeros((AGG_ROWS, D), jnp.float32)
    b1bc = jnp.broadcast_to(b1.reshape(L, 1, 2 * D), (L, 8, 2 * D))
    b2bc = jnp.broadcast_to(b2.reshape(L, 1, D), (L, 8, D))
    bpbc = jnp.broadcast_to(bp.reshape(1, C), (8, C))

    gidx, dstl, cnts = _partition(src, dst, attr)
    h = _embed(x3d, node_emb)
    for l in range(L):
        m2 = _mtable(h, edge_embs[l])
        agg = _aggregate(m2, gidx, dstl, cnts, zeros_hbm)
        epbc = jnp.full((8, 128), 1.0 + eps[l], jnp.float32)
        h = _mlp(h, agg, W1[l], b1bc[l], W2[l], b2bc[l], epbc, last=(l == L - 1))
    return _readout(h, batch3d, P, Wp, bpbc)


# speedup vs baseline: 4.4393x; 1.0399x over previous
import functools

import jax
import jax.numpy as jnp
from jax import lax
from jax.experimental import pallas as pl
from jax.experimental.pallas import tpu as pltpu
from jax.experimental.pallas import tpu_sc as plsc

N = 50000
E = 800000
D = 64
K = 4
C = 10
G = 64
GAMMA = 0.01
NV = 100
EV = 5

NW = 32
BLK = 1568
NPAD = NW * BLK
HALF = NPAD // 2
EPW = E // NW
PC = 512
NCH_P = (EPW + PC - 1) // PC
CAP = 25600
CH = 192
AGG_ROWS = 26112
TRASH = HALF
HI = lax.Precision.HIGHEST

_mesh = plsc.VectorSubcoreMesh(core_axis_name="c", subcore_axis_name="s")


def _vlast(v):
    return lax.squeeze(lax.slice(v, (15,), (16,)), (0,))


@functools.partial(
    pl.kernel,
    out_type=(
        jax.ShapeDtypeStruct((NW, 2, CAP), jnp.int32),
        jax.ShapeDtypeStruct((NW, 2, CAP), jnp.int32),
        jax.ShapeDtypeStruct((NW, 16), jnp.int32),
    ),
    mesh=_mesh,
    compiler_params=pltpu.CompilerParams(needs_layout_passes=False),
    scratch_types=[
        pltpu.VMEM((PC,), jnp.int32),
        pltpu.VMEM((PC,), jnp.int32),
        pltpu.VMEM((PC,), jnp.int32),
        pltpu.VMEM((CAP,), jnp.int32),
        pltpu.VMEM((CAP,), jnp.int32),
        pltpu.VMEM((CAP,), jnp.int32),
        pltpu.VMEM((CAP,), jnp.int32),
        pltpu.VMEM((16,), jnp.int32),
    ],
)
def _partition(src_hbm, dst_hbm, attr_hbm, gidx_out, dstl_out, cnt_out,
               src_v, dst_v, attr_v, g0, d0, g1, d1, cnt_v):
    c = lax.axis_index("c")
    s = lax.axis_index("s")
    w = s * 2 + c
    base = w * EPW
    lane = lax.broadcasted_iota(jnp.int32, (16,), 0)

    def chunk_body(g, offs):
        off = base + g * PC
        pltpu.sync_copy(src_hbm.at[pl.ds(off, PC)], src_v)
        pltpu.sync_copy(dst_hbm.at[pl.ds(off, PC)], dst_v)
        pltpu.sync_copy(attr_hbm.at[pl.ds(off, PC)], attr_v)

        npad16 = jnp.full((16,), NPAD, jnp.int32)
        half16 = jnp.full((16,), HALF, jnp.int32)
        epw16 = jnp.full((16,), EPW, jnp.int32)
        one16 = jnp.full((16,), 1, jnp.int32)
        zro16 = jnp.full((16,), 0, jnp.int32)

        def vec_body(i, offs2):
            off0, off1 = offs2
            s16 = src_v[pl.ds(i * 16, 16)]
            d16 = dst_v[pl.ds(i * 16, 16)]
            a16 = attr_v[pl.ds(i * 16, 16)]
            gidx = a16 * npad16 + s16
            pvec = lax.broadcast_in_dim(g * PC + i * 16, (16,), ()) + lane
            valid = pvec < epw16
            m0 = valid & (d16 < half16)
            m1 = valid & (d16 >= half16)
            mi0 = jnp.where(m0, one16, zro16)
            mi1 = jnp.where(m1, one16, zro16)
            cs0 = plsc.cumsum(mi0)
            cs1 = plsc.cumsum(mi1)
            pos0 = lax.broadcast_in_dim(off0, (16,), ()) + cs0 - mi0
            pos1 = lax.broadcast_in_dim(off1, (16,), ()) + cs1 - mi1
            plsc.store_scatter(g0, [pos0], gidx, mask=m0)
            plsc.store_scatter(d0, [pos0], d16, mask=m0)
            plsc.store_scatter(g1, [pos1], gidx, mask=m1)
            plsc.store_scatter(d1, [pos1], d16 - half16, mask=m1)
            return (off0 + _vlast(cs0), off1 + _vlast(cs1))

        return lax.fori_loop(0, PC // 16, vec_body, offs)

    off0, off1 = lax.fori_loop(0, NCH_P, chunk_body,
                               (jnp.int32(0), jnp.int32(0)))

    zero16 = jnp.zeros((16,), jnp.int32)
    trash16 = jnp.full((16,), TRASH, jnp.int32)
    for j in range(CH // 16):
        g0[pl.ds(off0 + j * 16, 16)] = zero16
        d0[pl.ds(off0 + j * 16, 16)] = trash16
        g1[pl.ds(off1 + j * 16, 16)] = zero16
        d1[pl.ds(off1 + j * 16, 16)] = trash16

    pltpu.sync_copy(g0, gidx_out.at[w, 0])
    pltpu.sync_copy(d0, dstl_out.at[w, 0])
    pltpu.sync_copy(g1, gidx_out.at[w, 1])
    pltpu.sync_copy(d1, dstl_out.at[w, 1])
    zv = jnp.zeros((16,), jnp.int32)
    cnt_v[...] = jnp.where(lane == zv, lax.broadcast_in_dim(off0, (16,), ()),
                           jnp.where(lane == zv + 1,
                                     lax.broadcast_in_dim(off1, (16,), ()), zv))
    pltpu.sync_copy(cnt_v, cnt_out.at[w])


@functools.partial(
    pl.kernel,
    out_type=jax.ShapeDtypeStruct((NPAD, D), jnp.float32),
    mesh=_mesh,
    compiler_params=pltpu.CompilerParams(needs_layout_passes=False,
                                         use_tc_tiling_on_sc=False),
    scratch_types=[
        pltpu.VMEM_SHARED((AGG_ROWS, D), jnp.float32),
        pltpu.VMEM((CH, D), jnp.float32),
        pltpu.VMEM((CH, D), jnp.float32),
        pltpu.VMEM((CH,), jnp.int32),
        pltpu.VMEM((CH,), jnp.int32),
        pltpu.VMEM((CH,), jnp.int32),
        pltpu.VMEM((CH,), jnp.int32),
        pltpu.VMEM((16,), jnp.int32),
        pltpu.SemaphoreType.DMA,
        pltpu.SemaphoreType.DMA,
        pltpu.SemaphoreType.DMA,
        pltpu.SemaphoreType.DMA,
    ],
)
def _aggregate(m_hbm, gidx_hbm, dstl_hbm, cnt_hbm, zeros_hbm, agg_out,
               agg_sh, bufa, bufb, gia, gib, da, db, cnt_v,
               sga, sgb, sia, sib):
    c = lax.axis_index("c")
    s = lax.axis_index("s")
    lane = lax.broadcasted_iota(jnp.int32, (16,), 0)

    pltpu.sync_copy(zeros_hbm.at[pl.ds(s * (AGG_ROWS // 16), AGG_ROWS // 16)],
                    agg_sh.at[pl.ds(s * (AGG_ROWS // 16), AGG_ROWS // 16)])
    plsc.subcore_barrier()

    for k in range(2):
        w = s * 2 + k
        pltpu.sync_copy(cnt_hbm.at[w], cnt_v)
        cvec = lax.broadcast_in_dim(c, (16,), ())
        sel = jnp.where(lane == cvec, cnt_v[...], jnp.zeros((16,), jnp.int32))
        cnt = _vlast(plsc.cumsum(sel))
        nch = (cnt + (CH - 1)) // CH
        hi = jnp.maximum(nch - 1, 0)

        def clamp(ci):
            return jnp.clip(ci, 0, hi)

        def idx_fetch(ci, gbuf_i, dbuf, si):
            cc = clamp(ci)
            pltpu.async_copy(gidx_hbm.at[w, c, pl.ds(cc * CH, CH)], gbuf_i, si)
            pltpu.async_copy(dstl_hbm.at[w, c, pl.ds(cc * CH, CH)], dbuf, si)

        def idx_wait(gbuf_i, dbuf, si):
            pltpu.make_async_copy(gidx_hbm.at[w, c, pl.ds(0, CH)],
                                  gbuf_i, si).wait()
            pltpu.make_async_copy(dstl_hbm.at[w, c, pl.ds(0, CH)],
                                  dbuf, si).wait()

        def gather_wait(buf, sg):
            pltpu.make_async_copy(m_hbm.at[gia], buf, sg).wait()

        idx_fetch(jnp.int32(0), gia, da, sia)
        idx_wait(gia, da, sia)
        pltpu.async_copy(m_hbm.at[gia], bufa, sga)
        idx_fetch(jnp.int32(1), gib, db, sib)

        def pair_body(g, _):
            c0 = 2 * g
            c1 = c0 + 1
            idx_wait(gib, db, sib)
            pltpu.async_copy(m_hbm.at[gib], bufb, sgb)
            gather_wait(bufa, sga)

            @pl.when(c0 < nch)
            def _():
                pltpu.sync_copy(bufa, agg_sh.at[da], add=True)

            idx_fetch(c0 + 2, gia, da, sia)
            idx_wait(gia, da, sia)
            pltpu.async_copy(m_hbm.at[gia], bufa, sga)
            gather_wait(bufb, sgb)

            @pl.when(c1 < nch)
            def _():
                pltpu.sync_copy(bufb, agg_sh.at[db], add=True)

            idx_fetch(c1 + 2, gib, db, sib)
            return 0

        lax.fori_loop(0, (nch + 1) // 2, pair_body, 0)
        gather_wait(bufa, sga)
        idx_wait(gib, db, sib)

    plsc.subcore_barrier()
    pltpu.sync_copy(agg_sh.at[pl.ds(s * BLK, BLK)],
                    agg_out.at[pl.ds(c * HALF + s * BLK, BLK)])


def _embed_body(x_ref, emb_ref, o_ref):
    xv = x_ref[...].reshape(1, BLK)
    oh = (lax.broadcasted_iota(jnp.int32, (NV, BLK), 0) == xv).astype(jnp.float32)
    o_ref[...] = lax.dot_general(oh, emb_ref[...], (((0,), (0,)), ((), ())),
                                 precision=HI)


def _embed(x3d, node_emb):
    return pl.pallas_call(
        _embed_body,
        grid=(NW,),
        in_specs=[
            pl.BlockSpec((1, 1, BLK), lambda b: (b, 0, 0)),
            pl.BlockSpec((NV, D), lambda b: (0, 0)),
        ],
        out_specs=pl.BlockSpec((BLK, D), lambda b: (b, 0)),
        out_shape=jax.ShapeDtypeStruct((NPAD, D), jnp.float32),
    )(x3d, node_emb)


def _mtable_body(h_ref, e_ref, o_ref):
    a = pl.program_id(0)
    sel = (lax.broadcasted_iota(jnp.int32, (EV, 1), 0) == a).astype(jnp.float32)
    erow = jnp.sum(e_ref[...] * sel, axis=0, keepdims=True)
    o_ref[...] = jnp.maximum(h_ref[...] + erow, 0.0)[None]


def _mtable(h, eemb):
    m3 = pl.pallas_call(
        _mtable_body,
        grid=(EV, NW),
        in_specs=[
            pl.BlockSpec((BLK, D), lambda a, b: (b, 0)),
            pl.BlockSpec((EV, D), lambda a, b: (0, 0)),
        ],
        out_specs=pl.BlockSpec((1, BLK, D), lambda a, b: (a, b, 0)),
        out_shape=jax.ShapeDtypeStruct((EV, NPAD, D), jnp.float32),
    )(h, eemb)
    return m3.reshape(EV * NPAD, D)


def _mlp_body(h_ref, a_ref, w1_ref, b1_ref, w2_ref, b2_ref, ep_ref, o_ref,
              *, last):
    z = h_ref[...] * ep_ref[0, 0] + a_ref[...]
    z1 = lax.dot_general(z, w1_ref[...], (((1,), (1,)), ((), ())),
                         precision=HI) + b1_ref[0:1, :]
    z1 = jnp.maximum(z1, 0.0)
    z2 = lax.dot_general(z1, w2_ref[...], (((1,), (1,)), ((), ())),
                         precision=HI) + b2_ref[0:1, :]
    o_ref[...] = z2 if last else jnp.maximum(z2, 0.0)


def _mlp(h, agg, w1, b1bc, w2, b2bc, epbc, last):
    return pl.pallas_call(
        functools.partial(_mlp_body, last=last),
        grid=(NW,),
        in_specs=[
            pl.BlockSpec((BLK, D), lambda b: (b, 0)),
            pl.BlockSpec((BLK, D), lambda b: (b, 0)),
            pl.BlockSpec((2 * D, D), lambda b: (0, 0)),
            pl.BlockSpec((8, 2 * D), lambda b: (0, 0)),
            pl.BlockSpec((D, 2 * D), lambda b: (0, 0)),
            pl.BlockSpec((8, D), lambda b: (0, 0)),
            pl.BlockSpec((8, 128), lambda b: (0, 0)),
        ],
        out_specs=pl.BlockSpec((BLK, D), lambda b: (b, 0)),
        out_shape=jax.ShapeDtypeStruct((NPAD, D), jnp.float32),
    )(h, agg, w1, b1bc, w2, b2bc, epbc)


def _readout_body(h_ref, bt_ref, p_ref, wp_ref, bp_ref, o_ref):
    b = pl.program_id(0)
    hh = h_ref[...]
    t = lax.dot_general(hh, p_ref[...], (((1,), (1,)), ((), ())),
                        precision=HI) * (1.0 / GAMMA)
    t = t - jnp.max(t, axis=1, keepdims=True)
    ex = jnp.exp(t)
    al = ex / jnp.sum(ex, axis=1, keepdims=True)
    w2 = jnp.concatenate([al[:, k:k + 1] * hh for k in range(K)], axis=1)
    ws = lax.dot_general(w2, wp_ref[...], (((1,), (1,)), ((), ())),
                         precision=HI)
    bv = bt_ref[...].reshape(1, BLK)
    oh = (lax.broadcasted_iota(jnp.int32, (G, BLK), 0) == bv).astype(jnp.float32)
    contrib = lax.dot_general(oh, ws, (((1,), (0,)), ((), ())), precision=HI)

    @pl.when(b == 0)
    def _():
        o_ref[...] = contrib + bp_ref[0:1, :]

    @pl.when(b > 0)
    def _():
        o_ref[...] = o_ref[...] + contrib


def _readout(h, batch3d, p, wp, bpbc):
    return pl.pallas_call(
        _readout_body,
        grid=(NW,),
        in_specs=[
            pl.BlockSpec((BLK, D), lambda b: (b, 0)),
            pl.BlockSpec((1, 1, BLK), lambda b: (b, 0, 0)),
            pl.BlockSpec((K, D), lambda b: (0, 0)),
            pl.BlockSpec((C, K * D), lambda b: (0, 0)),
            pl.BlockSpec((8, C), lambda b: (0, 0)),
        ],
        out_specs=pl.BlockSpec((G, C), lambda b: (0, 0)),
        out_shape=jax.ShapeDtypeStruct((G, C), jnp.float32),
    )(h, batch3d, p, wp, bpbc)


def kernel(x, edge_index, edge_attr, batch, node_emb, edge_embs,
           W1, b1, W2, b2, eps, P, Wp, bp):
    L = W1.shape[0]
    pad_e = NCH_P * PC - EPW
    src = jnp.pad(edge_index[0].astype(jnp.int32), (0, pad_e))
    dst = jnp.pad(edge_index[1].astype(jnp.int32), (0, pad_e))
    attr = jnp.pad(edge_attr.astype(jnp.int32), (0, pad_e))
    x3d = jnp.pad(x.astype(jnp.int32), (0, NPAD - N)).reshape(NW, 1, BLK)
    batch3d = jnp.pad(batch.astype(jnp.int32), (0, NPAD - N),
                      constant_values=G).reshape(NW, 1, BLK)
    zeros_hbm = jnp.zeros((AGG_ROWS, D), jnp.float32)
    b1bc = jnp.broadcast_to(b1.reshape(L, 1, 2 * D), (L, 8, 2 * D))
    b2bc = jnp.broadcast_to(b2.reshape(L, 1, D), (L, 8, D))
    bpbc = jnp.broadcast_to(bp.reshape(1, C), (8, C))

    gidx, dstl, cnts = _partition(src, dst, attr)
    h = _embed(x3d, node_emb)
    for l in range(L):
        m2 = _mtable(h, edge_embs[l])
        agg = _aggregate(m2, gidx, dstl, cnts, zeros_hbm)
        epbc = jnp.full((8, 128), 1.0 + eps[l], jnp.float32)
        h = _mlp(h, agg, W1[l], b1bc[l], W2[l], b2bc[l], epbc, last=(l == L - 1))
    return _readout(h, batch3d, P, Wp, bpbc)


# speedup vs baseline: 5.1792x; 1.1667x over previous
import functools

import jax
import jax.numpy as jnp
from jax import lax
from jax.experimental import pallas as pl
from jax.experimental.pallas import tpu as pltpu
from jax.experimental.pallas import tpu_sc as plsc

N = 50000
E = 800000
D = 64
K = 4
C = 10
G = 64
GAMMA = 0.01
NV = 100
EV = 5

NW = 32
BLK = 1568
NPAD = NW * BLK
HALF = NPAD // 2
EPW = E // NW
PC = 512
NCH_P = (EPW + PC - 1) // PC
CAP = 25600
CH = 192
AGG_ROWS = 26112
TRASH = HALF
HI = lax.Precision.HIGHEST

_mesh = plsc.VectorSubcoreMesh(core_axis_name="c", subcore_axis_name="s")


def _vlast(v):
    return lax.squeeze(lax.slice(v, (15,), (16,)), (0,))


@functools.partial(
    pl.kernel,
    out_type=(
        jax.ShapeDtypeStruct((NW, 2, CAP), jnp.int32),
        jax.ShapeDtypeStruct((NW, 2, CAP), jnp.int32),
        jax.ShapeDtypeStruct((NW, 16), jnp.int32),
    ),
    mesh=_mesh,
    compiler_params=pltpu.CompilerParams(needs_layout_passes=False),
    scratch_types=[
        pltpu.VMEM((PC,), jnp.int32),
        pltpu.VMEM((PC,), jnp.int32),
        pltpu.VMEM((PC,), jnp.int32),
        pltpu.VMEM((CAP,), jnp.int32),
        pltpu.VMEM((CAP,), jnp.int32),
        pltpu.VMEM((CAP,), jnp.int32),
        pltpu.VMEM((CAP,), jnp.int32),
        pltpu.VMEM((16,), jnp.int32),
    ],
)
def _partition(src_hbm, dst_hbm, attr_hbm, gidx_out, dstl_out, cnt_out,
               src_v, dst_v, attr_v, g0, d0, g1, d1, cnt_v):
    c = lax.axis_index("c")
    s = lax.axis_index("s")
    w = s * 2 + c
    base = w * EPW
    lane = lax.broadcasted_iota(jnp.int32, (16,), 0)

    def chunk_body(g, offs):
        off = base + g * PC
        pltpu.sync_copy(src_hbm.at[pl.ds(off, PC)], src_v)
        pltpu.sync_copy(dst_hbm.at[pl.ds(off, PC)], dst_v)
        pltpu.sync_copy(attr_hbm.at[pl.ds(off, PC)], attr_v)

        npad16 = jnp.full((16,), NPAD, jnp.int32)
        half16 = jnp.full((16,), HALF, jnp.int32)
        epw16 = jnp.full((16,), EPW, jnp.int32)
        one16 = jnp.full((16,), 1, jnp.int32)
        zro16 = jnp.full((16,), 0, jnp.int32)

        def vec_body(i, offs2):
            off0, off1 = offs2
            s16 = src_v[pl.ds(i * 16, 16)]
            d16 = dst_v[pl.ds(i * 16, 16)]
            a16 = attr_v[pl.ds(i * 16, 16)]
            gidx = a16 * npad16 + s16
            pvec = lax.broadcast_in_dim(g * PC + i * 16, (16,), ()) + lane
            valid = pvec < epw16
            m0 = valid & (d16 < half16)
            m1 = valid & (d16 >= half16)
            mi0 = jnp.where(m0, one16, zro16)
            mi1 = jnp.where(m1, one16, zro16)
            cs0 = plsc.cumsum(mi0)
            cs1 = plsc.cumsum(mi1)
            pos0 = lax.broadcast_in_dim(off0, (16,), ()) + cs0 - mi0
            pos1 = lax.broadcast_in_dim(off1, (16,), ()) + cs1 - mi1
            plsc.store_scatter(g0, [pos0], gidx, mask=m0)
            plsc.store_scatter(d0, [pos0], d16, mask=m0)
            plsc.store_scatter(g1, [pos1], gidx, mask=m1)
            plsc.store_scatter(d1, [pos1], d16 - half16, mask=m1)
            return (off0 + _vlast(cs0), off1 + _vlast(cs1))

        return lax.fori_loop(0, PC // 16, vec_body, offs)

    off0, off1 = lax.fori_loop(0, NCH_P, chunk_body,
                               (jnp.int32(0), jnp.int32(0)))

    zero16 = jnp.zeros((16,), jnp.int32)
    trash16 = jnp.full((16,), TRASH, jnp.int32)
    for j in range(CH // 16):
        g0[pl.ds(off0 + j * 16, 16)] = zero16
        d0[pl.ds(off0 + j * 16, 16)] = trash16
        g1[pl.ds(off1 + j * 16, 16)] = zero16
        d1[pl.ds(off1 + j * 16, 16)] = trash16

    pltpu.sync_copy(g0, gidx_out.at[w, 0])
    pltpu.sync_copy(d0, dstl_out.at[w, 0])
    pltpu.sync_copy(g1, gidx_out.at[w, 1])
    pltpu.sync_copy(d1, dstl_out.at[w, 1])
    zv = jnp.zeros((16,), jnp.int32)
    cnt_v[...] = jnp.where(lane == zv, lax.broadcast_in_dim(off0, (16,), ()),
                           jnp.where(lane == zv + 1,
                                     lax.broadcast_in_dim(off1, (16,), ()), zv))
    pltpu.sync_copy(cnt_v, cnt_out.at[w])


@functools.partial(
    pl.kernel,
    out_type=jax.ShapeDtypeStruct((NPAD, D), jnp.float32),
    mesh=_mesh,
    compiler_params=pltpu.CompilerParams(needs_layout_passes=False,
                                         use_tc_tiling_on_sc=False),
    scratch_types=[
        pltpu.VMEM_SHARED((AGG_ROWS, D), jnp.float32),
        pltpu.VMEM((CH, D), jnp.float32),
        pltpu.VMEM((CH, D), jnp.float32),
        pltpu.VMEM((CH,), jnp.int32),
        pltpu.VMEM((CH,), jnp.int32),
        pltpu.VMEM((CH,), jnp.int32),
        pltpu.VMEM((CH,), jnp.int32),
        pltpu.VMEM((16,), jnp.int32),
        pltpu.SemaphoreType.DMA,
        pltpu.SemaphoreType.DMA,
        pltpu.SemaphoreType.DMA,
        pltpu.SemaphoreType.DMA,
    ],
)
def _aggregate(m_hbm, gidx_hbm, dstl_hbm, cnt_hbm, zeros_hbm, agg_out,
               agg_sh, bufa, bufb, gia, gib, da, db, cnt_v,
               sga, sgb, sia, sib):
    c = lax.axis_index("c")
    s = lax.axis_index("s")
    lane = lax.broadcasted_iota(jnp.int32, (16,), 0)

    pltpu.sync_copy(zeros_hbm.at[pl.ds(s * (AGG_ROWS // 16), AGG_ROWS // 16)],
                    agg_sh.at[pl.ds(s * (AGG_ROWS // 16), AGG_ROWS // 16)])
    plsc.subcore_barrier()

    for k in range(2):
        w = s * 2 + k
        pltpu.sync_copy(cnt_hbm.at[w], cnt_v)
        cvec = lax.broadcast_in_dim(c, (16,), ())
        sel = jnp.where(lane == cvec, cnt_v[...], jnp.zeros((16,), jnp.int32))
        cnt = _vlast(plsc.cumsum(sel))
        nch = (cnt + (CH - 1)) // CH
        hi = jnp.maximum(nch - 1, 0)

        def clamp(ci):
            return jnp.clip(ci, 0, hi)

        def idx_fetch(ci, gbuf_i, dbuf, si):
            cc = clamp(ci)
            pltpu.async_copy(gidx_hbm.at[w, c, pl.ds(cc * CH, CH)], gbuf_i, si)
            pltpu.async_copy(dstl_hbm.at[w, c, pl.ds(cc * CH, CH)], dbuf, si)

        def idx_wait(gbuf_i, dbuf, si):
            pltpu.make_async_copy(gidx_hbm.at[w, c, pl.ds(0, CH)],
                                  gbuf_i, si).wait()
            pltpu.make_async_copy(dstl_hbm.at[w, c, pl.ds(0, CH)],
                                  dbuf, si).wait()

        def gather_wait(buf, sg):
            pltpu.make_async_copy(m_hbm.at[gia], buf, sg).wait()

        idx_fetch(jnp.int32(0), gia, da, sia)
        idx_wait(gia, da, sia)
        pltpu.async_copy(m_hbm.at[gia], bufa, sga)
        idx_fetch(jnp.int32(1), gib, db, sib)

        def pair_body(g, _):
            c0 = 2 * g
            c1 = c0 + 1
            idx_wait(gib, db, sib)
            pltpu.async_copy(m_hbm.at[gib], bufb, sgb)
            gather_wait(bufa, sga)

            @pl.when(c0 < nch)
            def _():
                pltpu.sync_copy(bufa, agg_sh.at[da], add=True)

            idx_fetch(c0 + 2, gia, da, sia)
            idx_wait(gia, da, sia)
            pltpu.async_copy(m_hbm.at[gia], bufa, sga)
            gather_wait(bufb, sgb)

            @pl.when(c1 < nch)
            def _():
                pltpu.sync_copy(bufb, agg_sh.at[db], add=True)

            idx_fetch(c1 + 2, gib, db, sib)
            return 0

        lax.fori_loop(0, (nch + 1) // 2, pair_body, 0)
        gather_wait(bufa, sga)
        idx_wait(gib, db, sib)

    plsc.subcore_barrier()
    pltpu.sync_copy(agg_sh.at[pl.ds(s * BLK, BLK)],
                    agg_out.at[pl.ds(c * HALF + s * BLK, BLK)])


def _embed_body(x_ref, emb_ref, o_ref):
    xv = x_ref[...].reshape(1, BLK)
    oh = (lax.broadcasted_iota(jnp.int32, (NV, BLK), 0) == xv).astype(jnp.float32)
    o_ref[...] = lax.dot_general(oh, emb_ref[...], (((0,), (0,)), ((), ())),
                                 precision=HI)


def _embed(x3d, node_emb):
    return pl.pallas_call(
        _embed_body,
        grid=(NW,),
        in_specs=[
            pl.BlockSpec((1, 1, BLK), lambda b: (b, 0, 0)),
            pl.BlockSpec((NV, D), lambda b: (0, 0)),
        ],
        out_specs=pl.BlockSpec((BLK, D), lambda b: (b, 0)),
        out_shape=jax.ShapeDtypeStruct((NPAD, D), jnp.float32),
    )(x3d, node_emb)


def _mtable_body(h_ref, e_ref, o_ref):
    a = pl.program_id(0)
    sel = (lax.broadcasted_iota(jnp.int32, (EV, 1), 0) == a).astype(jnp.float32)
    erow = jnp.sum(e_ref[...] * sel, axis=0, keepdims=True)
    o_ref[...] = jnp.maximum(h_ref[...] + erow, 0.0)[None]


def _mtable(h, eemb):
    m3 = pl.pallas_call(
        _mtable_body,
        grid=(EV, NW),
        in_specs=[
            pl.BlockSpec((BLK, D), lambda a, b: (b, 0)),
            pl.BlockSpec((EV, D), lambda a, b: (0, 0)),
        ],
        out_specs=pl.BlockSpec((1, BLK, D), lambda a, b: (a, b, 0)),
        out_shape=jax.ShapeDtypeStruct((EV, NPAD, D), jnp.float32),
    )(h, eemb)
    return m3.reshape(EV * NPAD, D)


def _mlp_body(h_ref, a_ref, w1_ref, b1_ref, w2_ref, b2_ref, ep_ref, o_ref,
              *, last):
    z = h_ref[...] * ep_ref[0, 0] + a_ref[...]
    z1 = lax.dot_general(z, w1_ref[...], (((1,), (1,)), ((), ()))) + b1_ref[0:1, :]
    z1 = jnp.maximum(z1, 0.0)
    z2 = lax.dot_general(z1, w2_ref[...], (((1,), (1,)), ((), ()))) + b2_ref[0:1, :]
    o_ref[...] = z2 if last else jnp.maximum(z2, 0.0)


def _mlp(h, agg, w1, b1bc, w2, b2bc, epbc, last):
    return pl.pallas_call(
        functools.partial(_mlp_body, last=last),
        grid=(NW,),
        in_specs=[
            pl.BlockSpec((BLK, D), lambda b: (b, 0)),
            pl.BlockSpec((BLK, D), lambda b: (b, 0)),
            pl.BlockSpec((2 * D, D), lambda b: (0, 0)),
            pl.BlockSpec((8, 2 * D), lambda b: (0, 0)),
            pl.BlockSpec((D, 2 * D), lambda b: (0, 0)),
            pl.BlockSpec((8, D), lambda b: (0, 0)),
            pl.BlockSpec((8, 128), lambda b: (0, 0)),
        ],
        out_specs=pl.BlockSpec((BLK, D), lambda b: (b, 0)),
        out_shape=jax.ShapeDtypeStruct((NPAD, D), jnp.float32),
    )(h, agg, w1, b1bc, w2, b2bc, epbc)


def _readout_body(h_ref, bt_ref, p_ref, wp_ref, bp_ref, o_ref, acc_ref):
    b = pl.program_id(0)
    hh = h_ref[...]
    t = lax.dot_general(hh, p_ref[...], (((1,), (1,)), ((), ()))) / GAMMA
    t = t - jnp.max(t, axis=1, keepdims=True)
    ex = jnp.exp(t)
    al = ex / jnp.sum(ex, axis=1, keepdims=True)
    w2 = jnp.concatenate([al[:, k:k + 1] * hh for k in range(K)], axis=1)
    bv = bt_ref[...].reshape(1, BLK)
    oh = (lax.broadcasted_iota(jnp.int32, (G, BLK), 0) == bv).astype(jnp.float32)
    contrib = lax.dot_general(oh, w2, (((1,), (0,)), ((), ())), precision=HI)

    @pl.when(b == 0)
    def _():
        acc_ref[...] = contrib

    @pl.when(b > 0)
    def _():
        acc_ref[...] = acc_ref[...] + contrib

    @pl.when(b == NW - 1)
    def _():
        o_ref[...] = lax.dot_general(acc_ref[...], wp_ref[...],
                                     (((1,), (1,)), ((), ()))) + bp_ref[0:1, :]


def _readout(h, batch3d, p, wp, bpbc):
    return pl.pallas_call(
        _readout_body,
        grid=(NW,),
        in_specs=[
            pl.BlockSpec((BLK, D), lambda b: (b, 0)),
            pl.BlockSpec((1, 1, BLK), lambda b: (b, 0, 0)),
            pl.BlockSpec((K, D), lambda b: (0, 0)),
            pl.BlockSpec((C, K * D), lambda b: (0, 0)),
            pl.BlockSpec((8, C), lambda b: (0, 0)),
        ],
        out_specs=pl.BlockSpec((G, C), lambda b: (0, 0)),
        out_shape=jax.ShapeDtypeStruct((G, C), jnp.float32),
        scratch_shapes=[pltpu.VMEM((G, K * D), jnp.float32)],
    )(h, batch3d, p, wp, bpbc)


def kernel(x, edge_index, edge_attr, batch, node_emb, edge_embs,
           W1, b1, W2, b2, eps, P, Wp, bp):
    L = W1.shape[0]
    pad_e = NCH_P * PC - EPW
    src = jnp.pad(edge_index[0].astype(jnp.int32), (0, pad_e))
    dst = jnp.pad(edge_index[1].astype(jnp.int32), (0, pad_e))
    attr = jnp.pad(edge_attr.astype(jnp.int32), (0, pad_e))
    x3d = jnp.pad(x.astype(jnp.int32), (0, NPAD - N)).reshape(NW, 1, BLK)
    batch3d = jnp.pad(batch.astype(jnp.int32), (0, NPAD - N),
                      constant_values=G).reshape(NW, 1, BLK)
    zeros_hbm = jnp.zeros((AGG_ROWS, D), jnp.float32)
    b1bc = jnp.broadcast_to(b1.reshape(L, 1, 2 * D), (L, 8, 2 * D))
    b2bc = jnp.broadcast_to(b2.reshape(L, 1, D), (L, 8, D))
    bpbc = jnp.broadcast_to(bp.reshape(1, C), (8, C))

    gidx, dstl, cnts = _partition(src, dst, attr)
    h = _embed(x3d, node_emb)
    for l in range(L):
        m2 = _mtable(h, edge_embs[l])
        agg = _aggregate(m2, gidx, dstl, cnts, zeros_hbm)
        epbc = jnp.full((8, 128), 1.0 + eps[l], jnp.float32)
        h = _mlp(h, agg, W1[l], b1bc[l], W2[l], b2bc[l], epbc, last=(l == L - 1))
    return _readout(h, batch3d, P, Wp, bpbc)
